# baseline (speedup 1.0000x reference)
import functools

import jax
import jax.numpy as jnp
from jax.experimental import pallas as pl
from jax.experimental.pallas import tpu as pltpu

_C = ((0.5, 0.25), (0.75, -0.25))
_BLOCK_ROWS = 2000


def _body(in0_ref, in1_ref, out_ref):
    in0 = in0_ref[...]
    in1 = in1_ref[...]

    k_t = jax.lax.broadcasted_iota(jnp.int32, (32, 256), 0)
    l_t = jax.lax.broadcasted_iota(jnp.int32, (32, 256), 1)
    same_t = (l_t // 128) == (k_t // 16)
    same_u = ((l_t % 128) // 8) == (k_t % 16)
    scale = jnp.where(k_t // 16 == 0, _C[0][0], _C[1][0]).astype(jnp.float32)
    B = jnp.where(same_t & same_u, scale, 0.0)

    r_i = jax.lax.broadcasted_iota(jnp.int32, (256, 16), 0)
    c_i = jax.lax.broadcasted_iota(jnp.int32, (256, 16), 1)
    S = jnp.where(
        ((r_i // 128) == (c_i // 8)) & ((r_i % 8) == (c_i % 8)), 1.0, 0.0
    ).astype(jnp.float32)

    in1a = in1[:, :128]
    in1b = in1[:, 128:]
    m0 = in1a + (_C[0][1] / _C[0][0]) * in1b
    m1 = in1a + (_C[1][1] / _C[1][0]) * in1b
    m = jnp.concatenate([m0, m1], axis=1)

    w = jax.lax.dot(in0, B, precision=jax.lax.Precision.HIGHEST)
    out_ref[...] = jax.lax.dot(
        w * m, S, precision=jax.lax.Precision.HIGHEST
    )


@jax.jit
def kernel(in0, in1):
    n = in0.shape[0]
    r = _BLOCK_ROWS
    grid = (pl.cdiv(n, r),)
    return pl.pallas_call(
        _body,
        grid=grid,
        in_specs=[
            pl.BlockSpec((r, 32), lambda i: (i, 0)),
            pl.BlockSpec((r, 256), lambda i: (i, 0)),
        ],
        out_specs=pl.BlockSpec((r, 16), lambda i: (i, 0)),
        out_shape=jax.ShapeDtypeStruct((n, 16), in0.dtype),
        compiler_params=pltpu.CompilerParams(
            dimension_semantics=("arbitrary",),
        ),
    )(in0, in1)


# speedup vs baseline: 2.6511x; 2.6511x over previous
import functools

import jax
import jax.numpy as jnp
from jax.experimental import pallas as pl
from jax.experimental.pallas import tpu as pltpu

_C = ((0.5, 0.25), (0.75, -0.25))
_BLOCK_ROWS = 2000


def _body(in0_ref, in1_ref, out_ref):
    in0 = in0_ref[...]
    in1 = in1_ref[...]

    k_t = jax.lax.broadcasted_iota(jnp.int32, (32, 256), 0)
    l_t = jax.lax.broadcasted_iota(jnp.int32, (32, 256), 1)
    same_t = (l_t // 128) == (k_t // 16)
    same_u = ((l_t % 128) // 8) == (k_t % 16)
    scale = jnp.where(k_t // 16 == 0, _C[0][0], _C[1][0]).astype(jnp.float32)
    B = jnp.where(same_t & same_u, scale, 0.0)

    r_i = jax.lax.broadcasted_iota(jnp.int32, (256, 16), 0)
    c_i = jax.lax.broadcasted_iota(jnp.int32, (256, 16), 1)
    S = jnp.where(
        ((r_i // 128) == (c_i // 8)) & ((r_i % 8) == (c_i % 8)), 1.0, 0.0
    ).astype(jnp.float32)

    in1a = in1[:, :128]
    in1b = in1[:, 128:]
    m0 = in1a + (_C[0][1] / _C[0][0]) * in1b
    m1 = in1a + (_C[1][1] / _C[1][0]) * in1b
    m = jnp.concatenate([m0, m1], axis=1)

    w = jax.lax.dot(in0, B, precision=jax.lax.Precision.DEFAULT)
    out_ref[...] = jax.lax.dot(
        w * m, S, precision=jax.lax.Precision.DEFAULT
    )


@jax.jit
def kernel(in0, in1):
    n = in0.shape[0]
    r = _BLOCK_ROWS
    grid = (pl.cdiv(n, r),)
    return pl.pallas_call(
        _body,
        grid=grid,
        in_specs=[
            pl.BlockSpec((r, 32), lambda i: (i, 0)),
            pl.BlockSpec((r, 256), lambda i: (i, 0)),
        ],
        out_specs=pl.BlockSpec((r, 16), lambda i: (i, 0)),
        out_shape=jax.ShapeDtypeStruct((n, 16), in0.dtype),
        compiler_params=pltpu.CompilerParams(
            dimension_semantics=("arbitrary",),
        ),
    )(in0, in1)
